# baseline (speedup 1.0000x reference)
import jax
import jax.numpy as jnp
from jax.experimental import pallas as pl
from jax.experimental.pallas import tpu as pltpu

B, T, D = 1, 2048, 768
NQ, NKV, DH = 12, 4, 64
BLK, STRIDE, TOPN, WIN = 32, 16, 16, 512
REP = NQ // NKV
NB = (T - BLK) // STRIDE + 1
NC = T // STRIDE
NBP = 128
SCALE = DH ** -0.5
NEG = -1e9

HI = jax.lax.Precision.HIGHEST
DEF = jax.lax.Precision.DEFAULT

PROJ_COLS = 3 * NQ + 6 * NKV * DH
PROJ_PAD = 1664

_INTERPRET = False


def _proj_body(x_ref, w_ref, b_ref, o_ref):
    y = jnp.dot(x_ref[...], w_ref[...], precision=DEF,
                preferred_element_type=jnp.float32) + b_ref[...]
    col = jax.lax.broadcasted_iota(jnp.int32, y.shape, 1)
    o_ref[...] = jnp.where(col < 3 * NQ, jax.nn.sigmoid(y), y)


def _run_proj(x2d, wcat, bcat):
    tq = 256
    return pl.pallas_call(
        _proj_body,
        grid=(T // tq,),
        in_specs=[
            pl.BlockSpec((tq, D), lambda i: (i, 0)),
            pl.BlockSpec((D, PROJ_PAD), lambda i: (0, 0)),
            pl.BlockSpec((1, PROJ_PAD), lambda i: (0, 0)),
        ],
        out_specs=pl.BlockSpec((tq, PROJ_PAD), lambda i: (i, 0)),
        out_shape=jax.ShapeDtypeStruct((T, PROJ_PAD), jnp.float32),
        interpret=_INTERPRET,
    )(x2d, wcat, bcat)


def _summ_body(kf_ref, vf_ref, bp_ref, k1_ref, k1b_ref, k2_ref, k2b_ref,
               v1_ref, v1b_ref, v2_ref, v2b_ref, ko_ref, vo_ref):
    bp = bp_ref[...]
    def mlp(f, w1, b1, w2, b2):
        h = jnp.dot(f + bp, w1, precision=DEF, preferred_element_type=jnp.float32) + b1
        h = 0.5 * h * (1.0 + jax.lax.erf(h * (2.0 ** -0.5)))
        return jnp.dot(h, w2, precision=DEF, preferred_element_type=jnp.float32) + b2
    ko_ref[...] = mlp(kf_ref[...], k1_ref[...], k1b_ref[...], k2_ref[...], k2b_ref[...])
    vo_ref[...] = mlp(vf_ref[...], v1_ref[...], v1b_ref[...], v2_ref[...], v2b_ref[...])


def _run_summ(kf, vf, bp_flat, ck1_w, ck1_b, ck2_w, ck2_b, cv1_w, cv1_b, cv2_w, cv2_b):
    n = kf.shape[0]
    def full(s):
        return pl.BlockSpec(s, lambda *_: tuple(0 for _ in s))
    return pl.pallas_call(
        _summ_body,
        in_specs=[full((n, BLK * DH)), full((n, BLK * DH)), full((1, BLK * DH)),
                  full((BLK * DH, DH)), full((1, DH)), full((DH, DH)), full((1, DH)),
                  full((BLK * DH, DH)), full((1, DH)), full((DH, DH)), full((1, DH))],
        out_specs=[full((n, DH)), full((n, DH))],
        out_shape=[jax.ShapeDtypeStruct((n, DH), jnp.float32),
                   jax.ShapeDtypeStruct((n, DH), jnp.float32)],
        interpret=_INTERPRET,
    )(kf, vf, bp_flat, ck1_w, ck1_b.reshape(1, DH), ck2_w, ck2_b.reshape(1, DH),
      cv1_w, cv1_b.reshape(1, DH), cv2_w, cv2_b.reshape(1, DH))


def _cmp_body(q_ref, ks_ref, vs_ref, oc_ref, cm_ref):
    qm = q_ref[...].reshape(REP * T, DH)
    ks = ks_ref[...].reshape(NBP, DH)
    s = jnp.dot(qm, ks.T, precision=DEF, preferred_element_type=jnp.float32) * SCALE
    rowt = jax.lax.broadcasted_iota(jnp.int32, (REP * T, NBP), 0) % T
    coln = jax.lax.broadcasted_iota(jnp.int32, (REP * T, NBP), 1)
    vis = (coln * STRIDE <= rowt) & (coln < NB)
    s = jnp.where(vis, s, NEG)
    m = jnp.max(s, axis=1, keepdims=True)
    e = jnp.exp(s - m)
    p = e / jnp.sum(e, axis=1, keepdims=True)
    oc = jnp.dot(p, vs_ref[...].reshape(NBP, DH), precision=DEF,
                 preferred_element_type=jnp.float32)
    oc_ref[...] = oc.reshape(REP, 1, T, DH)

    pg = jnp.sum(p.reshape(REP, T, NBP), axis=0)
    ci = jax.lax.broadcasted_iota(jnp.int32, (T, NBP), 1)
    w = jnp.where(ci < NB, pg, -1.0)
    sel = jnp.zeros((T, NBP), jnp.bool_)
    for _ in range(TOPN):
        wv = jnp.where(sel, -2.0, w)
        mx = jnp.max(wv, axis=1, keepdims=True)
        cand = wv == mx
        mi = jnp.min(jnp.where(cand, ci, NBP), axis=1, keepdims=True)
        sel = sel | (ci == mi)
    selm = sel.astype(jnp.float32)
    rr = jax.lax.broadcasted_iota(jnp.int32, (NBP, NBP), 0)
    cc = jax.lax.broadcasted_iota(jnp.int32, (NBP, NBP), 1)
    cover = ((cc == rr) | (cc == rr + 1)).astype(jnp.float32)
    cm_ref[...] = jnp.dot(selm, cover,
                          preferred_element_type=jnp.float32).reshape(1, T, NC)


def _run_cmp(qg, ksum_t, vsum_t):
    return pl.pallas_call(
        _cmp_body,
        grid=(NKV,),
        in_specs=[
            pl.BlockSpec((REP, 1, T, DH), lambda k: (0, k, 0, 0)),
            pl.BlockSpec((1, NBP, DH), lambda k: (k, 0, 0)),
            pl.BlockSpec((1, NBP, DH), lambda k: (k, 0, 0)),
        ],
        out_specs=[
            pl.BlockSpec((REP, 1, T, DH), lambda k: (0, k, 0, 0)),
            pl.BlockSpec((1, T, NC), lambda k: (k, 0, 0)),
        ],
        out_shape=[jax.ShapeDtypeStruct((REP, NKV, T, DH), jnp.float32),
                   jax.ShapeDtypeStruct((NKV, T, NC), jnp.float32)],
        interpret=_INTERPRET,
    )(qg, ksum_t, vsum_t)


def _swin_body(q_ref, ksl_ref, vsl_ref, kwn_ref, vwn_ref, cm_ref, g_ref,
               oc_ref, out_ref):
    i = pl.program_id(1)
    tq = q_ref.shape[2]
    q0 = i * tq
    qm = q_ref[...].reshape(REP * tq, DH)
    qs = (qm * SCALE).astype(jnp.bfloat16)

    ksl = ksl_ref[...].reshape(T, DH)
    s = jnp.dot(qs, ksl.T, precision=DEF, preferred_element_type=jnp.float32)
    cmf = cm_ref[...].reshape(tq, NC).astype(jnp.bfloat16)
    ccc = jax.lax.broadcasted_iota(jnp.int32, (NC, T), 0)
    jjj = jax.lax.broadcasted_iota(jnp.int32, (NC, T), 1)
    expand = (jjj // STRIDE == ccc).astype(jnp.bfloat16)
    selm = jnp.dot(cmf, expand, preferred_element_type=jnp.float32) > 0.5
    tloc = jax.lax.broadcasted_iota(jnp.int32, (tq, T), 0) + q0
    jloc = jax.lax.broadcasted_iota(jnp.int32, (tq, T), 1)
    mask = selm & (jloc <= tloc)
    s3 = s.reshape(REP, tq, T)
    s3 = jnp.where(mask[None], s3, NEG)
    m = jnp.max(s3, axis=2, keepdims=True)
    e = jnp.exp(s3 - m)
    p = (e / jnp.sum(e, axis=2, keepdims=True)).reshape(REP * tq, T)
    o_slc = jnp.dot(p.astype(jnp.bfloat16), vsl_ref[...].reshape(T, DH),
                    precision=DEF, preferred_element_type=jnp.float32)

    wb = WIN + tq
    start = pl.multiple_of(jnp.clip(q0 - WIN, 0, T - wb), tq)
    kw = kwn_ref[0, pl.ds(start, wb), :]
    vw = vwn_ref[0, pl.ds(start, wb), :]
    sw = jnp.dot(qs, kw.T, precision=DEF, preferred_element_type=jnp.float32)
    tl2 = jax.lax.broadcasted_iota(jnp.int32, (tq, wb), 0) + q0
    jl2 = jax.lax.broadcasted_iota(jnp.int32, (tq, wb), 1) + start
    dist = tl2 - jl2
    wmask = (dist >= 0) & (dist < WIN)
    sw3 = sw.reshape(REP, tq, wb)
    sw3 = jnp.where(wmask[None], sw3, NEG)
    mw = jnp.max(sw3, axis=2, keepdims=True)
    ew = jnp.exp(sw3 - mw)
    pw = (ew / jnp.sum(ew, axis=2, keepdims=True)).reshape(REP * tq, wb)
    o_win = jnp.dot(pw.astype(jnp.bfloat16), vw, precision=DEF,
                    preferred_element_type=jnp.float32)

    g = g_ref[...].reshape(REP, tq, 3)
    o_cmp = oc_ref[...].reshape(REP, tq, DH)
    out = (g[..., 0:1] * o_cmp
           + g[..., 1:2] * o_slc.reshape(REP, tq, DH)
           + g[..., 2:3] * o_win.reshape(REP, tq, DH))
    out_ref[...] = out.reshape(REP, 1, tq, DH)


def _run_swin(qg, k_slc_t, v_slc_t, k_win_t, v_win_t, cm, gates, o_cmp):
    tq = 512
    return pl.pallas_call(
        _swin_body,
        grid=(NKV, T // tq),
        in_specs=[
            pl.BlockSpec((REP, 1, tq, DH), lambda k, i: (0, k, i, 0)),
            pl.BlockSpec((1, T, DH), lambda k, i: (k, 0, 0)),
            pl.BlockSpec((1, T, DH), lambda k, i: (k, 0, 0)),
            pl.BlockSpec((1, T, DH), lambda k, i: (k, 0, 0)),
            pl.BlockSpec((1, T, DH), lambda k, i: (k, 0, 0)),
            pl.BlockSpec((1, tq, NC), lambda k, i: (k, i, 0)),
            pl.BlockSpec((1, REP, tq, 3), lambda k, i: (k, 0, i, 0)),
            pl.BlockSpec((REP, 1, tq, DH), lambda k, i: (0, k, i, 0)),
        ],
        out_specs=pl.BlockSpec((REP, 1, tq, DH), lambda k, i: (0, k, i, 0)),
        out_shape=jax.ShapeDtypeStruct((REP, NKV, T, DH), jnp.float32),
        interpret=_INTERPRET,
    )(qg, k_slc_t, v_slc_t, k_win_t, v_win_t, cm, gates, o_cmp)


def kernel(x, q, Wg, bg, Wk_cmp, Wv_cmp, Wk_slc, Wv_slc, Wk_win, Wv_win,
           block_pos, ck1_w, ck1_b, ck2_w, ck2_b, cv1_w, cv1_b, cv2_w, cv2_b):
    x2d = x.reshape(T, D)
    wcat = jnp.concatenate([Wg, Wk_cmp, Wv_cmp, Wk_slc, Wv_slc, Wk_win, Wv_win],
                           axis=1)
    wcat = jnp.pad(wcat, ((0, 0), (0, PROJ_PAD - PROJ_COLS)))
    bcat = jnp.pad(bg, (0, PROJ_PAD - 3 * NQ)).reshape(1, PROJ_PAD)
    proj = _run_proj(x2d, wcat, bcat)

    gall = proj[:, :3 * NQ].reshape(T, REP, NKV, 3)
    gates = gall.transpose(2, 1, 0, 3)
    c0 = 3 * NQ
    w = NKV * DH

    def grab(idx):
        return proj[:, c0 + idx * w: c0 + (idx + 1) * w].reshape(T, NKV, DH)

    k_cmp, v_cmp = grab(0), grab(1)
    bf = jnp.bfloat16
    k_slc_t = grab(2).transpose(1, 0, 2).astype(bf)
    v_slc_t = grab(3).transpose(1, 0, 2).astype(bf)
    k_win_t = grab(4).transpose(1, 0, 2).astype(bf)
    v_win_t = grab(5).transpose(1, 0, 2).astype(bf)

    def blockify(kv):
        ch = kv.reshape(NC, STRIDE, NKV, DH)
        blk = jnp.concatenate([ch[:NB], ch[1:NB + 1]], axis=1)
        flat = blk.transpose(0, 2, 1, 3).reshape(NB * NKV, BLK * DH)
        return jnp.pad(flat, ((0, (NBP - NB) * NKV, ), (0, 0)))

    kf, vf = blockify(k_cmp), blockify(v_cmp)
    bp_flat = block_pos.reshape(1, BLK * DH)
    ksum, vsum = _run_summ(kf, vf, bp_flat, ck1_w, ck1_b, ck2_w, ck2_b,
                           cv1_w, cv1_b, cv2_w, cv2_b)
    ksum_t = ksum.reshape(NBP, NKV, DH).transpose(1, 0, 2)
    vsum_t = vsum.reshape(NBP, NKV, DH).transpose(1, 0, 2)

    qg = q.reshape(REP, NKV, T, DH)
    o_cmp, cm = _run_cmp(qg, ksum_t, vsum_t)
    out = _run_swin(qg, k_slc_t, v_slc_t, k_win_t, v_win_t, cm, gates, o_cmp)
    return out.reshape(B, NQ, T, DH)


# speedup vs baseline: 1.1148x; 1.1148x over previous
import jax
import jax.numpy as jnp
from jax.experimental import pallas as pl
from jax.experimental.pallas import tpu as pltpu

B, T, D = 1, 2048, 768
NQ, NKV, DH = 12, 4, 64
BLK, STRIDE, TOPN, WIN = 32, 16, 16, 512
REP = NQ // NKV
NB = (T - BLK) // STRIDE + 1
NC = T // STRIDE
NBP = 128
SCALE = DH ** -0.5
NEG = -1e9

HI = jax.lax.Precision.HIGHEST
DEF = jax.lax.Precision.DEFAULT

PROJ_COLS = 3 * NQ + 6 * NKV * DH
PROJ_PAD = 1664

_INTERPRET = False


def _proj_body(x_ref, w_ref, b_ref, o_ref):
    y = jnp.dot(x_ref[...], w_ref[...], precision=DEF,
                preferred_element_type=jnp.float32) + b_ref[...]
    col = jax.lax.broadcasted_iota(jnp.int32, y.shape, 1)
    o_ref[...] = jnp.where(col < 3 * NQ, jax.nn.sigmoid(y), y)


def _run_proj(x2d, wcat, bcat):
    tq = 256
    return pl.pallas_call(
        _proj_body,
        grid=(T // tq,),
        in_specs=[
            pl.BlockSpec((tq, D), lambda i: (i, 0)),
            pl.BlockSpec((D, PROJ_PAD), lambda i: (0, 0)),
            pl.BlockSpec((1, PROJ_PAD), lambda i: (0, 0)),
        ],
        out_specs=pl.BlockSpec((tq, PROJ_PAD), lambda i: (i, 0)),
        out_shape=jax.ShapeDtypeStruct((T, PROJ_PAD), jnp.float32),
        interpret=_INTERPRET,
    )(x2d, wcat, bcat)


def _summ_body(kf_ref, vf_ref, bp_ref, k1_ref, k1b_ref, k2_ref, k2b_ref,
               v1_ref, v1b_ref, v2_ref, v2b_ref, ko_ref, vo_ref):
    bp = bp_ref[...]
    def mlp(f, w1, b1, w2, b2):
        h = jnp.dot(f + bp, w1, precision=DEF, preferred_element_type=jnp.float32) + b1
        h = 0.5 * h * (1.0 + jax.lax.erf(h * (2.0 ** -0.5)))
        return jnp.dot(h, w2, precision=DEF, preferred_element_type=jnp.float32) + b2
    ko_ref[...] = mlp(kf_ref[...], k1_ref[...], k1b_ref[...], k2_ref[...], k2b_ref[...])
    vo_ref[...] = mlp(vf_ref[...], v1_ref[...], v1b_ref[...], v2_ref[...], v2b_ref[...])


def _run_summ(kf, vf, bp_flat, ck1_w, ck1_b, ck2_w, ck2_b, cv1_w, cv1_b, cv2_w, cv2_b):
    n = kf.shape[0]
    def full(s):
        return pl.BlockSpec(s, lambda *_: tuple(0 for _ in s))
    return pl.pallas_call(
        _summ_body,
        in_specs=[full((n, BLK * DH)), full((n, BLK * DH)), full((1, BLK * DH)),
                  full((BLK * DH, DH)), full((1, DH)), full((DH, DH)), full((1, DH)),
                  full((BLK * DH, DH)), full((1, DH)), full((DH, DH)), full((1, DH))],
        out_specs=[full((n, DH)), full((n, DH))],
        out_shape=[jax.ShapeDtypeStruct((n, DH), jnp.float32),
                   jax.ShapeDtypeStruct((n, DH), jnp.float32)],
        interpret=_INTERPRET,
    )(kf, vf, bp_flat, ck1_w, ck1_b.reshape(1, DH), ck2_w, ck2_b.reshape(1, DH),
      cv1_w, cv1_b.reshape(1, DH), cv2_w, cv2_b.reshape(1, DH))


def _cmp_body(q_ref, ks_ref, vs_ref, oc_ref, cm_ref):
    qm = q_ref[...].reshape(REP * T, DH)
    ks = ks_ref[...].reshape(NBP, DH)
    s = jnp.dot(qm, ks.T, precision=DEF, preferred_element_type=jnp.float32) * SCALE
    rowt = jax.lax.broadcasted_iota(jnp.int32, (REP * T, NBP), 0) % T
    coln = jax.lax.broadcasted_iota(jnp.int32, (REP * T, NBP), 1)
    vis = (coln * STRIDE <= rowt) & (coln < NB)
    s = jnp.where(vis, s, NEG)
    m = jnp.max(s, axis=1, keepdims=True)
    e = jnp.exp(s - m)
    p = e / jnp.sum(e, axis=1, keepdims=True)
    oc = jnp.dot(p, vs_ref[...].reshape(NBP, DH), precision=DEF,
                 preferred_element_type=jnp.float32)
    oc_ref[...] = oc.reshape(REP, 1, T, DH)

    pg = jnp.sum(p.reshape(REP, T, NBP), axis=0)
    ci = jax.lax.broadcasted_iota(jnp.int32, (T, NBP), 1)
    w = jnp.where(ci < NB, pg, -1.0)
    sel = jnp.zeros((T, NBP), jnp.bool_)
    for _ in range(TOPN):
        wv = jnp.where(sel, -2.0, w)
        mx = jnp.max(wv, axis=1, keepdims=True)
        cand = wv == mx
        mi = jnp.min(jnp.where(cand, ci, NBP), axis=1, keepdims=True)
        sel = sel | (ci == mi)
    selm = sel.astype(jnp.float32)
    rr = jax.lax.broadcasted_iota(jnp.int32, (NBP, NBP), 0)
    cc = jax.lax.broadcasted_iota(jnp.int32, (NBP, NBP), 1)
    cover = ((cc == rr) | (cc == rr + 1)).astype(jnp.float32)
    cm_ref[...] = jnp.dot(selm, cover,
                          preferred_element_type=jnp.float32).reshape(1, T, NC)


def _run_cmp(qg, ksum_t, vsum_t):
    return pl.pallas_call(
        _cmp_body,
        grid=(NKV,),
        in_specs=[
            pl.BlockSpec((REP, 1, T, DH), lambda k: (0, k, 0, 0)),
            pl.BlockSpec((1, NBP, DH), lambda k: (k, 0, 0)),
            pl.BlockSpec((1, NBP, DH), lambda k: (k, 0, 0)),
        ],
        out_specs=[
            pl.BlockSpec((REP, 1, T, DH), lambda k: (0, k, 0, 0)),
            pl.BlockSpec((1, T, NC), lambda k: (k, 0, 0)),
        ],
        out_shape=[jax.ShapeDtypeStruct((REP, NKV, T, DH), jnp.float32),
                   jax.ShapeDtypeStruct((NKV, T, NC), jnp.float32)],
        interpret=_INTERPRET,
    )(qg, ksum_t, vsum_t)


TQ4 = 512


def _make_swin_body(i):
    tq = TQ4
    q0 = i * tq
    kwid = q0 + tq
    woff = max(0, q0 - WIN)
    wwid = kwid - woff if q0 < WIN else WIN + tq

    def body(q_ref, ksl_ref, vsl_ref, kwn_ref, vwn_ref, cm_ref, g_ref,
             oc_ref, out_ref):
        qm = q_ref[...].reshape(REP * tq, DH)
        qs = qm * SCALE

        ksl = ksl_ref[0, :kwid, :]
        s = jnp.dot(qs, ksl.T, precision=DEF, preferred_element_type=jnp.float32)
        cmf = cm_ref[...].reshape(tq, NC)
        ccc = jax.lax.broadcasted_iota(jnp.int32, (NC, kwid), 0)
        jjj = jax.lax.broadcasted_iota(jnp.int32, (NC, kwid), 1)
        expand = (jjj // STRIDE == ccc).astype(jnp.float32)
        selm = jnp.dot(cmf, expand, preferred_element_type=jnp.float32) > 0.5
        tloc = jax.lax.broadcasted_iota(jnp.int32, (tq, kwid), 0) + q0
        jloc = jax.lax.broadcasted_iota(jnp.int32, (tq, kwid), 1)
        mask = selm & (jloc <= tloc)
        s3 = jnp.where(mask[None], s.reshape(REP, tq, kwid), NEG)
        m = jnp.max(s3, axis=2, keepdims=True)
        e = jnp.exp(s3 - m)
        p = (e / jnp.sum(e, axis=2, keepdims=True)).reshape(REP * tq, kwid)
        o_slc = jnp.dot(p, vsl_ref[0, :kwid, :], precision=DEF,
                        preferred_element_type=jnp.float32)

        kw = kwn_ref[0, woff:woff + wwid, :]
        vw = vwn_ref[0, woff:woff + wwid, :]
        sw = jnp.dot(qs, kw.T, precision=DEF, preferred_element_type=jnp.float32)
        tl2 = jax.lax.broadcasted_iota(jnp.int32, (tq, wwid), 0) + q0
        jl2 = jax.lax.broadcasted_iota(jnp.int32, (tq, wwid), 1) + woff
        dist = tl2 - jl2
        wmask = (dist >= 0) & (dist < WIN)
        sw3 = jnp.where(wmask[None], sw.reshape(REP, tq, wwid), NEG)
        mw = jnp.max(sw3, axis=2, keepdims=True)
        ew = jnp.exp(sw3 - mw)
        pw = (ew / jnp.sum(ew, axis=2, keepdims=True)).reshape(REP * tq, wwid)
        o_win = jnp.dot(pw, vw, precision=DEF, preferred_element_type=jnp.float32)

        g = g_ref[...].reshape(REP, tq, 3)
        o_cmp = oc_ref[...].reshape(REP, tq, DH)
        out = (g[..., 0:1] * o_cmp
               + g[..., 1:2] * o_slc.reshape(REP, tq, DH)
               + g[..., 2:3] * o_win.reshape(REP, tq, DH))
        out_ref[...] = out.reshape(REP, 1, tq, DH)

    return body


def _run_swin(qg, k_slc_t, v_slc_t, k_win_t, v_win_t, cm, gates, o_cmp):
    tq = TQ4
    outs = []
    for i in range(T // tq):
        out_i = pl.pallas_call(
            _make_swin_body(i),
            grid=(NKV,),
            in_specs=[
                pl.BlockSpec((REP, 1, tq, DH), lambda k, i=i: (0, k, i, 0)),
                pl.BlockSpec((1, T, DH), lambda k: (k, 0, 0)),
                pl.BlockSpec((1, T, DH), lambda k: (k, 0, 0)),
                pl.BlockSpec((1, T, DH), lambda k: (k, 0, 0)),
                pl.BlockSpec((1, T, DH), lambda k: (k, 0, 0)),
                pl.BlockSpec((1, tq, NC), lambda k, i=i: (k, i, 0)),
                pl.BlockSpec((1, REP, tq, 3), lambda k, i=i: (k, 0, i, 0)),
                pl.BlockSpec((REP, 1, tq, DH), lambda k, i=i: (0, k, i, 0)),
            ],
            out_specs=pl.BlockSpec((REP, 1, tq, DH), lambda k: (0, k, 0, 0)),
            out_shape=jax.ShapeDtypeStruct((REP, NKV, tq, DH), jnp.float32),
            interpret=_INTERPRET,
        )(qg, k_slc_t, v_slc_t, k_win_t, v_win_t, cm, gates, o_cmp)
        outs.append(out_i)
    return jnp.concatenate(outs, axis=2)


def kernel(x, q, Wg, bg, Wk_cmp, Wv_cmp, Wk_slc, Wv_slc, Wk_win, Wv_win,
           block_pos, ck1_w, ck1_b, ck2_w, ck2_b, cv1_w, cv1_b, cv2_w, cv2_b):
    x2d = x.reshape(T, D)
    wcat = jnp.concatenate([Wg, Wk_cmp, Wv_cmp, Wk_slc, Wv_slc, Wk_win, Wv_win],
                           axis=1)
    wcat = jnp.pad(wcat, ((0, 0), (0, PROJ_PAD - PROJ_COLS)))
    bcat = jnp.pad(bg, (0, PROJ_PAD - 3 * NQ)).reshape(1, PROJ_PAD)
    proj = _run_proj(x2d, wcat, bcat)

    gall = proj[:, :3 * NQ].reshape(T, REP, NKV, 3)
    gates = gall.transpose(2, 1, 0, 3)
    c0 = 3 * NQ
    w = NKV * DH

    def grab(idx):
        return proj[:, c0 + idx * w: c0 + (idx + 1) * w].reshape(T, NKV, DH)

    k_cmp, v_cmp = grab(0), grab(1)
    k_slc_t = grab(2).transpose(1, 0, 2)
    v_slc_t = grab(3).transpose(1, 0, 2)
    k_win_t = grab(4).transpose(1, 0, 2)
    v_win_t = grab(5).transpose(1, 0, 2)

    def blockify(kv):
        ch = kv.reshape(NC, STRIDE, NKV, DH)
        blk = jnp.concatenate([ch[:NB], ch[1:NB + 1]], axis=1)
        flat = blk.transpose(0, 2, 1, 3).reshape(NB * NKV, BLK * DH)
        return jnp.pad(flat, ((0, (NBP - NB) * NKV, ), (0, 0)))

    kf, vf = blockify(k_cmp), blockify(v_cmp)
    bp_flat = block_pos.reshape(1, BLK * DH)
    ksum, vsum = _run_summ(kf, vf, bp_flat, ck1_w, ck1_b, ck2_w, ck2_b,
                           cv1_w, cv1_b, cv2_w, cv2_b)
    ksum_t = ksum.reshape(NBP, NKV, DH).transpose(1, 0, 2)
    vsum_t = vsum.reshape(NBP, NKV, DH).transpose(1, 0, 2)

    qg = q.reshape(REP, NKV, T, DH)
    o_cmp, cm = _run_cmp(qg, ksum_t, vsum_t)
    out = _run_swin(qg, k_slc_t, v_slc_t, k_win_t, v_win_t, cm, gates, o_cmp)
    return out.reshape(B, NQ, T, DH)


# speedup vs baseline: 1.1659x; 1.0459x over previous
import jax
import jax.numpy as jnp
from jax.experimental import pallas as pl
from jax.experimental.pallas import tpu as pltpu

B, T, D = 1, 2048, 768
NQ, NKV, DH = 12, 4, 64
BLK, STRIDE, TOPN, WIN = 32, 16, 16, 512
REP = NQ // NKV
NB = (T - BLK) // STRIDE + 1
NC = T // STRIDE
NBP = 128
SCALE = DH ** -0.5
NEG = -1e9

HI = jax.lax.Precision.HIGHEST
DEF = jax.lax.Precision.DEFAULT

PROJ_COLS = 3 * NQ + 6 * NKV * DH
PROJ_PAD = 1664

_INTERPRET = False


def _proj_body(x_ref, w_ref, b_ref, o_ref):
    y = jnp.dot(x_ref[...], w_ref[...], precision=DEF,
                preferred_element_type=jnp.float32) + b_ref[...]
    col = jax.lax.broadcasted_iota(jnp.int32, y.shape, 1)
    o_ref[...] = jnp.where(col < 3 * NQ, jax.nn.sigmoid(y), y)


def _run_proj(x2d, wcat, bcat):
    tq = 256
    return pl.pallas_call(
        _proj_body,
        grid=(T // tq,),
        in_specs=[
            pl.BlockSpec((tq, D), lambda i: (i, 0)),
            pl.BlockSpec((D, PROJ_PAD), lambda i: (0, 0)),
            pl.BlockSpec((1, PROJ_PAD), lambda i: (0, 0)),
        ],
        out_specs=pl.BlockSpec((tq, PROJ_PAD), lambda i: (i, 0)),
        out_shape=jax.ShapeDtypeStruct((T, PROJ_PAD), jnp.float32),
        interpret=_INTERPRET,
    )(x2d, wcat, bcat)


def _summ_body(kf_ref, vf_ref, bp_ref, k1_ref, k1b_ref, k2_ref, k2b_ref,
               v1_ref, v1b_ref, v2_ref, v2b_ref, ko_ref, vo_ref):
    bp = bp_ref[...]
    def mlp(f, w1, b1, w2, b2):
        h = jnp.dot(f + bp, w1, precision=DEF, preferred_element_type=jnp.float32) + b1
        h = 0.5 * h * (1.0 + jax.lax.erf(h * (2.0 ** -0.5)))
        return jnp.dot(h, w2, precision=DEF, preferred_element_type=jnp.float32) + b2
    ko_ref[...] = mlp(kf_ref[...], k1_ref[...], k1b_ref[...], k2_ref[...], k2b_ref[...])
    vo_ref[...] = mlp(vf_ref[...], v1_ref[...], v1b_ref[...], v2_ref[...], v2b_ref[...])


def _run_summ(kf, vf, bp_flat, ck1_w, ck1_b, ck2_w, ck2_b, cv1_w, cv1_b, cv2_w, cv2_b):
    n = kf.shape[0]
    def full(s):
        return pl.BlockSpec(s, lambda *_: tuple(0 for _ in s))
    return pl.pallas_call(
        _summ_body,
        in_specs=[full((n, BLK * DH)), full((n, BLK * DH)), full((1, BLK * DH)),
                  full((BLK * DH, DH)), full((1, DH)), full((DH, DH)), full((1, DH)),
                  full((BLK * DH, DH)), full((1, DH)), full((DH, DH)), full((1, DH))],
        out_specs=[full((n, DH)), full((n, DH))],
        out_shape=[jax.ShapeDtypeStruct((n, DH), jnp.float32),
                   jax.ShapeDtypeStruct((n, DH), jnp.float32)],
        interpret=_INTERPRET,
    )(kf, vf, bp_flat, ck1_w, ck1_b.reshape(1, DH), ck2_w, ck2_b.reshape(1, DH),
      cv1_w, cv1_b.reshape(1, DH), cv2_w, cv2_b.reshape(1, DH))


def _cmp_body(q_ref, ks_ref, vs_ref, oc_ref, cm_ref):
    qm = q_ref[...].reshape(REP * T, DH)
    ks = ks_ref[...].reshape(NBP, DH)
    s = jnp.dot(qm, ks.T, precision=DEF, preferred_element_type=jnp.float32) * SCALE
    rowt = jax.lax.broadcasted_iota(jnp.int32, (REP * T, NBP), 0) % T
    coln = jax.lax.broadcasted_iota(jnp.int32, (REP * T, NBP), 1)
    vis = (coln * STRIDE <= rowt) & (coln < NB)
    s = jnp.where(vis, s, NEG)
    m = jnp.max(s, axis=1, keepdims=True)
    e = jnp.exp(s - m)
    p = e / jnp.sum(e, axis=1, keepdims=True)
    oc = jnp.dot(p, vs_ref[...].reshape(NBP, DH), precision=DEF,
                 preferred_element_type=jnp.float32)
    oc_ref[...] = oc.reshape(REP, 1, T, DH)

    pg = jnp.sum(p.reshape(REP, T, NBP), axis=0)
    ci = jax.lax.broadcasted_iota(jnp.int32, (T, NBP), 1)
    ik = jax.lax.bitcast_convert_type(pg, jnp.int32)
    key = jnp.where(ci < NB, (ik & ~127) | (127 - ci), jnp.int32(-2 ** 31))
    sel = jnp.zeros((T, NBP), jnp.bool_)
    for _ in range(TOPN):
        mx = jnp.max(key, axis=1, keepdims=True)
        oh = key == mx
        sel = sel | oh
        key = jnp.where(oh, jnp.int32(-2 ** 31), key)
    selm = sel.astype(jnp.float32)
    rr = jax.lax.broadcasted_iota(jnp.int32, (NBP, NBP), 0)
    cc = jax.lax.broadcasted_iota(jnp.int32, (NBP, NBP), 1)
    cover = ((cc == rr) | (cc == rr + 1)).astype(jnp.float32)
    cm_ref[...] = jnp.dot(selm, cover,
                          preferred_element_type=jnp.float32).reshape(1, T, NC)


def _run_cmp(qg, ksum_t, vsum_t):
    return pl.pallas_call(
        _cmp_body,
        grid=(NKV,),
        in_specs=[
            pl.BlockSpec((REP, 1, T, DH), lambda k: (0, k, 0, 0)),
            pl.BlockSpec((1, NBP, DH), lambda k: (k, 0, 0)),
            pl.BlockSpec((1, NBP, DH), lambda k: (k, 0, 0)),
        ],
        out_specs=[
            pl.BlockSpec((REP, 1, T, DH), lambda k: (0, k, 0, 0)),
            pl.BlockSpec((1, T, NC), lambda k: (k, 0, 0)),
        ],
        out_shape=[jax.ShapeDtypeStruct((REP, NKV, T, DH), jnp.float32),
                   jax.ShapeDtypeStruct((NKV, T, NC), jnp.float32)],
        interpret=_INTERPRET,
    )(qg, ksum_t, vsum_t)


TQ4 = 512


def _make_swin_body(i):
    tq = TQ4
    q0 = i * tq
    kwid = q0 + tq
    woff = max(0, q0 - WIN)
    wwid = kwid - woff if q0 < WIN else WIN + tq

    def body(q_ref, ksl_ref, vsl_ref, kwn_ref, vwn_ref, cm_ref, g_ref,
             oc_ref, out_ref):
        qm = q_ref[...].reshape(REP * tq, DH)
        qs = qm * SCALE

        ksl = ksl_ref[0, :kwid, :]
        s = jnp.dot(qs, ksl.T, precision=DEF, preferred_element_type=jnp.float32)
        cmf = cm_ref[...].reshape(tq, NC)
        ccc = jax.lax.broadcasted_iota(jnp.int32, (NC, kwid), 0)
        jjj = jax.lax.broadcasted_iota(jnp.int32, (NC, kwid), 1)
        expand = (jjj // STRIDE == ccc).astype(jnp.float32)
        selm = jnp.dot(cmf, expand, preferred_element_type=jnp.float32) > 0.5
        tloc = jax.lax.broadcasted_iota(jnp.int32, (tq, kwid), 0) + q0
        jloc = jax.lax.broadcasted_iota(jnp.int32, (tq, kwid), 1)
        mask = selm & (jloc <= tloc)
        s3 = jnp.where(mask[None], s.reshape(REP, tq, kwid), NEG)
        m = jnp.max(s3, axis=2, keepdims=True)
        e = jnp.exp(s3 - m)
        p = (e / jnp.sum(e, axis=2, keepdims=True)).reshape(REP * tq, kwid)
        o_slc = jnp.dot(p, vsl_ref[0, :kwid, :], precision=DEF,
                        preferred_element_type=jnp.float32)

        kw = kwn_ref[0, woff:woff + wwid, :]
        vw = vwn_ref[0, woff:woff + wwid, :]
        sw = jnp.dot(qs, kw.T, precision=DEF, preferred_element_type=jnp.float32)
        tl2 = jax.lax.broadcasted_iota(jnp.int32, (tq, wwid), 0) + q0
        jl2 = jax.lax.broadcasted_iota(jnp.int32, (tq, wwid), 1) + woff
        dist = tl2 - jl2
        wmask = (dist >= 0) & (dist < WIN)
        sw3 = jnp.where(wmask[None], sw.reshape(REP, tq, wwid), NEG)
        mw = jnp.max(sw3, axis=2, keepdims=True)
        ew = jnp.exp(sw3 - mw)
        pw = (ew / jnp.sum(ew, axis=2, keepdims=True)).reshape(REP * tq, wwid)
        o_win = jnp.dot(pw, vw, precision=DEF, preferred_element_type=jnp.float32)

        g = g_ref[...].reshape(REP, tq, 3)
        o_cmp = oc_ref[...].reshape(REP, tq, DH)
        out = (g[..., 0:1] * o_cmp
               + g[..., 1:2] * o_slc.reshape(REP, tq, DH)
               + g[..., 2:3] * o_win.reshape(REP, tq, DH))
        out_ref[...] = out.reshape(REP, 1, tq, DH)

    return body


def _run_swin(qg, k_slc_t, v_slc_t, k_win_t, v_win_t, cm, gates, o_cmp):
    tq = TQ4
    outs = []
    for i in range(T // tq):
        out_i = pl.pallas_call(
            _make_swin_body(i),
            grid=(NKV,),
            in_specs=[
                pl.BlockSpec((REP, 1, tq, DH), lambda k, i=i: (0, k, i, 0)),
                pl.BlockSpec((1, T, DH), lambda k: (k, 0, 0)),
                pl.BlockSpec((1, T, DH), lambda k: (k, 0, 0)),
                pl.BlockSpec((1, T, DH), lambda k: (k, 0, 0)),
                pl.BlockSpec((1, T, DH), lambda k: (k, 0, 0)),
                pl.BlockSpec((1, tq, NC), lambda k, i=i: (k, i, 0)),
                pl.BlockSpec((1, REP, tq, 3), lambda k, i=i: (k, 0, i, 0)),
                pl.BlockSpec((REP, 1, tq, DH), lambda k, i=i: (0, k, i, 0)),
            ],
            out_specs=pl.BlockSpec((REP, 1, tq, DH), lambda k: (0, k, 0, 0)),
            out_shape=jax.ShapeDtypeStruct((REP, NKV, tq, DH), jnp.float32),
            interpret=_INTERPRET,
        )(qg, k_slc_t, v_slc_t, k_win_t, v_win_t, cm, gates, o_cmp)
        outs.append(out_i)
    return jnp.concatenate(outs, axis=2)


def kernel(x, q, Wg, bg, Wk_cmp, Wv_cmp, Wk_slc, Wv_slc, Wk_win, Wv_win,
           block_pos, ck1_w, ck1_b, ck2_w, ck2_b, cv1_w, cv1_b, cv2_w, cv2_b):
    x2d = x.reshape(T, D)
    wcat = jnp.concatenate([Wg, Wk_cmp, Wv_cmp, Wk_slc, Wv_slc, Wk_win, Wv_win],
                           axis=1)
    wcat = jnp.pad(wcat, ((0, 0), (0, PROJ_PAD - PROJ_COLS)))
    bcat = jnp.pad(bg, (0, PROJ_PAD - 3 * NQ)).reshape(1, PROJ_PAD)
    proj = _run_proj(x2d, wcat, bcat)

    gall = proj[:, :3 * NQ].reshape(T, REP, NKV, 3)
    gates = gall.transpose(2, 1, 0, 3)
    c0 = 3 * NQ
    w = NKV * DH

    def grab(idx):
        return proj[:, c0 + idx * w: c0 + (idx + 1) * w].reshape(T, NKV, DH)

    k_cmp, v_cmp = grab(0), grab(1)
    k_slc_t = grab(2).transpose(1, 0, 2)
    v_slc_t = grab(3).transpose(1, 0, 2)
    k_win_t = grab(4).transpose(1, 0, 2)
    v_win_t = grab(5).transpose(1, 0, 2)

    def blockify(kv):
        ch = kv.reshape(NC, STRIDE, NKV, DH)
        blk = jnp.concatenate([ch[:NB], ch[1:NB + 1]], axis=1)
        flat = blk.transpose(0, 2, 1, 3).reshape(NB * NKV, BLK * DH)
        return jnp.pad(flat, ((0, (NBP - NB) * NKV, ), (0, 0)))

    kf, vf = blockify(k_cmp), blockify(v_cmp)
    bp_flat = block_pos.reshape(1, BLK * DH)
    ksum, vsum = _run_summ(kf, vf, bp_flat, ck1_w, ck1_b, ck2_w, ck2_b,
                           cv1_w, cv1_b, cv2_w, cv2_b)
    ksum_t = ksum.reshape(NBP, NKV, DH).transpose(1, 0, 2)
    vsum_t = vsum.reshape(NBP, NKV, DH).transpose(1, 0, 2)

    qg = q.reshape(REP, NKV, T, DH)
    o_cmp, cm = _run_cmp(qg, ksum_t, vsum_t)
    out = _run_swin(qg, k_slc_t, v_slc_t, k_win_t, v_win_t, cm, gates, o_cmp)
    return out.reshape(B, NQ, T, DH)
